# baseline (speedup 1.0000x reference)
import functools

import jax
import jax.numpy as jnp
from jax import lax
from jax.experimental import pallas as pl
from jax.experimental.pallas import tpu as pltpu
from jax.experimental.pallas import tpu_sc as plsc

_N_NODES = 10000
_N_EDGES = 320000
_D = 128
_F = 16

_NC = 2
_NS = 16
_NW = _NC * _NS
_E_TILE = _N_EDGES // _NW
_CHUNK = 128
_E_PAD = 10240
_NCHUNK = _E_PAD // _CHUNK
_ROWS_TILE = _N_NODES // _NS
_TRASH = _N_NODES


def _matmul_body(x_ref, w_ref, h_ref):
    h_ref[...] = jnp.dot(x_ref[...], w_ref[...],
                         preferred_element_type=jnp.float32)


def _softmax_body(p_ref, o_ref):
    s = p_ref[0] + p_ref[1]
    m = jnp.max(s, axis=-1, keepdims=True)
    e = jnp.exp(s - m)
    o_ref[...] = e / jnp.sum(e, axis=-1, keepdims=True)


def _agg_body(h_hbm, edge_hbm, ew_hbm, zero_hbm, out_hbm,
              sidx, didx, wv, msg0, msg1, msg2, msg3, acc,
              gsem0, gsem1, gsem2, gsem3, ssem0, ssem1, ssem2, ssem3):
    cid = lax.axis_index("c")
    sid = lax.axis_index("s")
    wid = cid * _NS + sid
    r0 = sid * _ROWS_TILE
    e0 = wid * _E_TILE

    z = pltpu.async_copy(zero_hbm, acc.at[pl.ds(r0, _ROWS_TILE)], gsem0)
    ls = pltpu.async_copy(edge_hbm.at[pl.ds(e0, _E_TILE)],
                          sidx.at[pl.ds(0, _E_TILE)], gsem1)
    ld = pltpu.async_copy(edge_hbm.at[pl.ds(_N_EDGES + e0, _E_TILE)],
                          didx.at[pl.ds(0, _E_TILE)], gsem2)
    lw = pltpu.async_copy(ew_hbm.at[pl.ds(e0, _E_TILE)],
                          wv.at[pl.ds(0, _E_TILE)], gsem3)
    for g in range((_E_PAD - _E_TILE) // 16):
        o = _E_TILE + g * 16
        sidx[pl.ds(o, 16)] = jnp.zeros((16,), jnp.int32)
        didx[pl.ds(o, 16)] = _TRASH + g * 16 + lax.iota(jnp.int32, 16)
        wv[pl.ds(o, 16)] = jnp.zeros((16,), jnp.float32)
    z.wait()
    ls.wait()
    ld.wait()
    lw.wait()
    plsc.subcore_barrier()

    def _scale(msg, k):
        for g in range(_CHUNK // 16):
            w16 = wv[pl.ds(k * _CHUNK + g * 16, 16)]
            for j in range(16):
                e = g * 16 + j
                msg[e, :] = msg[e, :] * w16[j]

    def _gather(k, msg, sem):
        pltpu.async_copy(h_hbm.at[sidx.at[pl.ds(k * _CHUNK, _CHUNK)]],
                         msg, sem)

    def _gwait(msg, sem):
        pltpu.make_async_copy(
            h_hbm.at[sidx.at[pl.ds(0, _CHUNK)]], msg, sem).wait()

    def _scatter(msg, k, sem):
        return pltpu.async_copy(
            msg, acc.at[didx.at[pl.ds(k * _CHUNK, _CHUNK)]], sem, add=True)

    msgs = (msg0, msg1, msg2, msg3)
    gsems = (gsem0, gsem1, gsem2, gsem3)
    ssems = (ssem0, ssem1, ssem2, ssem3)
    for b in range(4):
        _gather(b, msgs[b], gsems[b])

    _nit = _NCHUNK // 4

    def _ring(i, carry):
        k = 4 * i
        scats = []
        for b in range(4):
            _gwait(msgs[b], gsems[b])
            _scale(msgs[b], k + b)
            scats.append(_scatter(msgs[b], k + b, ssems[b]))
        for b in range(4):
            scats[b].wait()

            @pl.when(i < _nit - 1)
            def _():
                _gather(k + 4 + b, msgs[b], gsems[b])

        return carry

    lax.fori_loop(0, _nit, _ring, 0)

    plsc.subcore_barrier()
    pltpu.sync_copy(acc.at[pl.ds(r0, _ROWS_TILE)],
                    out_hbm.at[cid, pl.ds(r0, _ROWS_TILE)])


def kernel(x, edge_index, edge_weight, kernel):
    w = kernel
    edges = edge_index.astype(jnp.int32).reshape(-1)
    ew = edge_weight.astype(jnp.float32)
    zero = jnp.zeros((_ROWS_TILE, _F), jnp.float32)

    h = pl.pallas_call(
        _matmul_body,
        out_shape=jax.ShapeDtypeStruct((_N_NODES, _F), jnp.float32),
    )(x, w)

    mesh = plsc.VectorSubcoreMesh(core_axis_name="c", subcore_axis_name="s")
    agg_fn = functools.partial(
        pl.kernel,
        mesh=mesh,
        out_type=jax.ShapeDtypeStruct((_NC, _N_NODES, _F), jnp.float32),
        scratch_types=[
            pltpu.VMEM((_E_PAD,), jnp.int32),
            pltpu.VMEM((_E_PAD,), jnp.int32),
            pltpu.VMEM((_E_PAD,), jnp.float32),
            pltpu.VMEM((_CHUNK, _F), jnp.float32),
            pltpu.VMEM((_CHUNK, _F), jnp.float32),
            pltpu.VMEM((_CHUNK, _F), jnp.float32),
            pltpu.VMEM((_CHUNK, _F), jnp.float32),
            pltpu.VMEM_SHARED((_N_NODES + 240, _F), jnp.float32),
            pltpu.SemaphoreType.DMA,
            pltpu.SemaphoreType.DMA,
            pltpu.SemaphoreType.DMA,
            pltpu.SemaphoreType.DMA,
            pltpu.SemaphoreType.DMA,
            pltpu.SemaphoreType.DMA,
            pltpu.SemaphoreType.DMA,
            pltpu.SemaphoreType.DMA,
        ],
        compiler_params=pltpu.CompilerParams(use_tc_tiling_on_sc=False),
    )(_agg_body)
    parts = agg_fn(h, edges, ew, zero)

    out = pl.pallas_call(
        _softmax_body,
        out_shape=jax.ShapeDtypeStruct((_N_NODES, _F), jnp.float32),
    )(parts)
    return out


# speedup vs baseline: 1.3021x; 1.3021x over previous
import functools

import jax
import jax.numpy as jnp
from jax import lax
from jax.experimental import pallas as pl
from jax.experimental.pallas import tpu as pltpu
from jax.experimental.pallas import tpu_sc as plsc

_N_NODES = 10000
_N_EDGES = 320000
_D = 128
_F = 16

_NC = 2
_NS = 16
_NW = _NC * _NS
_E_TILE = _N_EDGES // _NW
_CHUNK = 80
_NCHUNK = _E_TILE // _CHUNK
_ROWS_TILE = _N_NODES // _NS


def _matmul_body(x_ref, w_ref, h_ref):
    h_ref[...] = jnp.dot(x_ref[...], w_ref[...],
                         preferred_element_type=jnp.float32)


def _softmax_body(p_ref, o_ref):
    s = p_ref[0] + p_ref[1]
    m = jnp.max(s, axis=-1, keepdims=True)
    e = jnp.exp(s - m)
    o_ref[...] = e / jnp.sum(e, axis=-1, keepdims=True)


def _agg_body(h_hbm, edge_hbm, ew_hbm, zero_hbm, out_hbm,
              sidx, didx, wv, msg0, msg1, msg2, msg3, acc,
              gsem0, gsem1, gsem2, gsem3, ssem0, ssem1, ssem2, ssem3):
    cid = lax.axis_index("c")
    sid = lax.axis_index("s")
    wid = cid * _NS + sid
    r0 = sid * _ROWS_TILE
    e0 = wid * _E_TILE

    pltpu.sync_copy(zero_hbm, acc.at[pl.ds(r0, _ROWS_TILE)])
    pltpu.sync_copy(edge_hbm.at[pl.ds(e0, _E_TILE)], sidx)
    pltpu.sync_copy(edge_hbm.at[pl.ds(_N_EDGES + e0, _E_TILE)], didx)
    pltpu.sync_copy(ew_hbm.at[pl.ds(e0, _E_TILE)], wv)
    plsc.subcore_barrier()

    def _scale(msg, k):
        for g in range(_CHUNK // 16):
            w16 = wv[pl.ds(k * _CHUNK + g * 16, 16)]
            for j in range(16):
                e = g * 16 + j
                msg[e, :] = msg[e, :] * w16[j]

    def _gather(k, msg, sem):
        pltpu.async_copy(h_hbm.at[sidx.at[pl.ds(k * _CHUNK, _CHUNK)]],
                         msg, sem)

    def _gwait(msg, sem):
        pltpu.make_async_copy(
            h_hbm.at[sidx.at[pl.ds(0, _CHUNK)]], msg, sem).wait()

    def _scatter(msg, k, sem):
        return pltpu.async_copy(
            msg, acc.at[didx.at[pl.ds(k * _CHUNK, _CHUNK)]], sem, add=True)

    msgs = (msg0, msg1, msg2, msg3)
    gsems = (gsem0, gsem1, gsem2, gsem3)
    ssems = (ssem0, ssem1, ssem2, ssem3)
    for b in range(4):
        _gather(b, msgs[b], gsems[b])

    _nit = (_NCHUNK - 1) // 4

    def _ring(i, carry):
        k = 4 * i
        scats = []
        for b in range(4):
            _gwait(msgs[b], gsems[b])
            _scale(msgs[b], k + b)
            scats.append(_scatter(msgs[b], k + b, ssems[b]))
        scats[0].wait()
        _gather(k + 4, msgs[0], gsems[0])
        for b in range(1, 4):
            scats[b].wait()

            @pl.when(i < _nit - 1)
            def _():
                _gather(k + 4 + b, msgs[b], gsems[b])

        return carry

    lax.fori_loop(0, _nit, _ring, 0)

    _gwait(msg0, gsem0)
    _scale(msg0, _NCHUNK - 1)
    pltpu.sync_copy(msg0, acc.at[didx.at[pl.ds((_NCHUNK - 1) * _CHUNK,
                                               _CHUNK)]], add=True)

    plsc.subcore_barrier()
    pltpu.sync_copy(acc.at[pl.ds(r0, _ROWS_TILE)],
                    out_hbm.at[cid, pl.ds(r0, _ROWS_TILE)])


def kernel(x, edge_index, edge_weight, kernel):
    w = kernel
    edges = edge_index.astype(jnp.int32).reshape(-1)
    ew = edge_weight.astype(jnp.float32)
    zero = jnp.zeros((_ROWS_TILE, _F), jnp.float32)

    h = pl.pallas_call(
        _matmul_body,
        out_shape=jax.ShapeDtypeStruct((_N_NODES, _F), jnp.float32),
    )(x, w)

    mesh = plsc.VectorSubcoreMesh(core_axis_name="c", subcore_axis_name="s")
    agg_fn = functools.partial(
        pl.kernel,
        mesh=mesh,
        out_type=jax.ShapeDtypeStruct((_NC, _N_NODES, _F), jnp.float32),
        scratch_types=[
            pltpu.VMEM((_E_TILE,), jnp.int32),
            pltpu.VMEM((_E_TILE,), jnp.int32),
            pltpu.VMEM((_E_TILE,), jnp.float32),
            pltpu.VMEM((_CHUNK, _F), jnp.float32),
            pltpu.VMEM((_CHUNK, _F), jnp.float32),
            pltpu.VMEM((_CHUNK, _F), jnp.float32),
            pltpu.VMEM((_CHUNK, _F), jnp.float32),
            pltpu.VMEM_SHARED((_N_NODES, _F), jnp.float32),
            pltpu.SemaphoreType.DMA,
            pltpu.SemaphoreType.DMA,
            pltpu.SemaphoreType.DMA,
            pltpu.SemaphoreType.DMA,
            pltpu.SemaphoreType.DMA,
            pltpu.SemaphoreType.DMA,
            pltpu.SemaphoreType.DMA,
            pltpu.SemaphoreType.DMA,
        ],
        compiler_params=pltpu.CompilerParams(use_tc_tiling_on_sc=False),
    )(_agg_body)
    parts = agg_fn(h, edges, ew, zero)

    out = pl.pallas_call(
        _softmax_body,
        out_shape=jax.ShapeDtypeStruct((_N_NODES, _F), jnp.float32),
    )(parts)
    return out
